# baseline (speedup 1.0000x reference)
import functools

import jax
import jax.numpy as jnp
from jax import lax
from jax.experimental import pallas as pl
from jax.experimental.pallas import tpu as pltpu
from jax.experimental.pallas import tpu_sc as plsc

N = 10000
E = 320000
C = 128

NC = 2
NS = 16
NW = NC * NS
EPW = E // NW
K = 64
EPP = 10240
CH = EPP // K
NBUF = 4
NP = 10112
RPT = NP // NS
NPH = 10240
RPTH = NPH // NS
SHIFT = 14
MASK = (1 << SHIFT) - 1

_mesh = plsc.VectorSubcoreMesh(core_axis_name="c", subcore_axis_name="s")


@functools.partial(
    pl.kernel,
    out_type=[
        jax.ShapeDtypeStruct((NPH,), jnp.float32),
        jax.ShapeDtypeStruct((NPH,), jnp.float32),
    ],
    mesh=_mesh,
    scratch_types=[
        pltpu.VMEM((CH, K), jnp.int32),
        pltpu.VMEM((K,), jnp.float32),
        pltpu.VMEM_SHARED((NPH,), jnp.float32),
    ],
)
def _hist_kernel(dst_hbm, ones_hbm, zeros_hbm, h0_hbm, h1_hbm,
                 idx_v, ones_v, hist_sh):
    c = lax.axis_index("c")
    s = lax.axis_index("s")
    w = c * NS + s
    sl = pl.ds(s * RPTH, RPTH)
    pltpu.sync_copy(zeros_hbm, hist_sh.at[sl])
    pltpu.sync_copy(dst_hbm.at[w], idx_v)
    pltpu.sync_copy(ones_hbm, ones_v)
    plsc.subcore_barrier()

    def body(j, carry):
        pltpu.sync_copy(ones_v, hist_sh.at[idx_v.at[j]], add=True)
        return carry

    lax.fori_loop(0, CH, body, 0)
    plsc.subcore_barrier()

    @pl.when(c == 0)
    def _():
        pltpu.sync_copy(hist_sh.at[sl], h0_hbm.at[sl])

    @pl.when(c == 1)
    def _():
        pltpu.sync_copy(hist_sh.at[sl], h1_hbm.at[sl])


@functools.partial(
    pl.kernel,
    out_type=[
        jax.ShapeDtypeStruct((NP, C), jnp.float32),
        jax.ShapeDtypeStruct((NP, C), jnp.float32),
    ],
    mesh=_mesh,
    scratch_types=(
        [pltpu.VMEM((EPP,), jnp.int32)]
        + [pltpu.VMEM((K,), jnp.int32) for _ in range(NBUF)]
        + [pltpu.VMEM((K,), jnp.int32) for _ in range(NBUF)]
        + [pltpu.VMEM((K, C), jnp.float32) for _ in range(NBUF)]
        + [pltpu.VMEM_SHARED((NP, C), jnp.float32)]
        + [pltpu.SemaphoreType.DMA for _ in range(NBUF)]
    ),
)
def _edge_kernel(hs_hbm, pidx_hbm, zeros_hbm, acc0_hbm, acc1_hbm,
                 pidx_v, *rest):
    c = lax.axis_index("c")
    s = lax.axis_index("s")
    w = c * NS + s
    sl = pl.ds(s * RPT, RPT)
    su = rest[0:NBUF]
    du = rest[NBUF:2 * NBUF]
    rows = rest[2 * NBUF:3 * NBUF]
    acc_sh = rest[3 * NBUF]
    gsem = rest[3 * NBUF + 1:4 * NBUF + 1]

    pltpu.sync_copy(zeros_hbm, acc_sh.at[sl])
    pltpu.sync_copy(pidx_hbm.at[w], pidx_v)
    plsc.subcore_barrier()

    def unpack(j, p):
        for t in range(K // 16):
            v = pidx_v[pl.ds(j * K + 16 * t, 16)]
            su[p][pl.ds(16 * t, 16)] = lax.shift_right_logical(v, SHIFT)
            du[p][pl.ds(16 * t, 16)] = lax.bitwise_and(v, MASK)

    for p in range(NBUF - 1):
        unpack(p, p)
        pltpu.async_copy(hs_hbm.at[su[p]], rows[p], gsem[p])

    def step(j, r, prefetch=True):
        pltpu.make_async_copy(hs_hbm.at[su[r]], rows[r], gsem[r]).wait()
        if prefetch:
            q = (r + NBUF - 1) % NBUF
            unpack(j + NBUF - 1, q)
            pltpu.async_copy(hs_hbm.at[su[q]], rows[q], gsem[q])
        pltpu.sync_copy(rows[r], acc_sh.at[du[r]], add=True)

    def body(jj, carry):
        for r in range(NBUF):
            step(NBUF * jj + r, r)
        return carry

    nloop = (CH - (NBUF - 1)) // NBUF
    lax.fori_loop(0, nloop, body, 0)
    for j in range(NBUF * nloop, CH):
        step(j, j % NBUF, prefetch=(j + NBUF - 1 < CH))

    plsc.subcore_barrier()

    @pl.when(c == 0)
    def _():
        pltpu.sync_copy(acc_sh.at[sl], acc0_hbm.at[sl])

    @pl.when(c == 1)
    def _():
        pltpu.sync_copy(acc_sh.at[sl], acc1_hbm.at[sl])


def _prep_body(x_ref, w_ref, h0_ref, h1_ref, hs_ref, dis_ref):
    deg = h0_ref[...] + h1_ref[...] + 1.0
    dis = lax.rsqrt(deg)
    h = jnp.dot(x_ref[...], w_ref[...], preferred_element_type=jnp.float32)
    hs_ref[...] = h * dis
    dis_ref[...] = dis


def _final_body(a0_ref, a1_ref, hs_ref, dis_ref, b_ref, out_ref):
    acc = a0_ref[...] + a1_ref[...] + hs_ref[...]
    out_ref[...] = jnp.maximum(acc * dis_ref[...] + b_ref[...], 0.0)


_RB = 1000


def _tc_prep(x, W, h0, h1):
    return pl.pallas_call(
        _prep_body,
        grid=(N // _RB,),
        in_specs=[
            pl.BlockSpec((_RB, C), lambda i: (i, 0)),
            pl.BlockSpec((C, C), lambda i: (0, 0)),
            pl.BlockSpec((_RB, 1), lambda i: (i, 0)),
            pl.BlockSpec((_RB, 1), lambda i: (i, 0)),
        ],
        out_specs=[
            pl.BlockSpec((_RB, C), lambda i: (i, 0)),
            pl.BlockSpec((_RB, 1), lambda i: (i, 0)),
        ],
        out_shape=[
            jax.ShapeDtypeStruct((N, C), jnp.float32),
            jax.ShapeDtypeStruct((N, 1), jnp.float32),
        ],
    )(x, W, h0, h1)


def _tc_final(a0, a1, hs, dis, b):
    return pl.pallas_call(
        _final_body,
        grid=(N // _RB,),
        in_specs=[
            pl.BlockSpec((_RB, C), lambda i: (i, 0)),
            pl.BlockSpec((_RB, C), lambda i: (i, 0)),
            pl.BlockSpec((_RB, C), lambda i: (i, 0)),
            pl.BlockSpec((_RB, 1), lambda i: (i, 0)),
            pl.BlockSpec((1, C), lambda i: (0, 0)),
        ],
        out_specs=pl.BlockSpec((_RB, C), lambda i: (i, 0)),
        out_shape=jax.ShapeDtypeStruct((N, C), jnp.float32),
    )(a0, a1, hs, dis, b)


def kernel(x, edge_index, W, b):
    pad = EPP - EPW
    src_w = jnp.concatenate(
        [edge_index[0].reshape(NW, EPW), jnp.zeros((NW, pad), jnp.int32)],
        axis=1)
    dum = N + (jnp.arange(pad, dtype=jnp.int32) % (NP - N))
    dst_w = jnp.concatenate(
        [edge_index[1].reshape(NW, EPW), jnp.broadcast_to(dum, (NW, pad))],
        axis=1)
    dst3d = dst_w.reshape(NW, CH, K)
    pidx2d = (lax.shift_left(src_w, SHIFT) | dst_w).reshape(NW, EPP)
    ones_k = jnp.ones((K,), jnp.float32)
    zeros1 = jnp.zeros((RPTH,), jnp.float32)
    zeros128 = jnp.zeros((RPT, C), jnp.float32)

    h0, h1 = _hist_kernel(dst3d, ones_k, zeros1)
    hs, dis = _tc_prep(x, W, h0.reshape(NPH, 1), h1.reshape(NPH, 1))
    acc0, acc1 = _edge_kernel(hs, pidx2d, zeros128)
    return _tc_final(acc0, acc1, hs, dis, b.reshape(1, C))


# speedup vs baseline: 2.6286x; 2.6286x over previous
import functools

import jax
import jax.numpy as jnp
from jax import lax
from jax.experimental import pallas as pl
from jax.experimental.pallas import tpu as pltpu
from jax.experimental.pallas import tpu_sc as plsc

N = 10000
E = 320000
C = 128

NC = 2
NS = 16
NW = NC * NS
EPW = E // NW
K = 80
CH = EPW // K
NP = 10240
RPT = NP // NS
SHIFT = 14
MASK = (1 << SHIFT) - 1

_mesh = plsc.VectorSubcoreMesh(core_axis_name="c", subcore_axis_name="s")


@functools.partial(
    pl.kernel,
    out_type=[
        jax.ShapeDtypeStruct((NP,), jnp.float32),
        jax.ShapeDtypeStruct((NP,), jnp.float32),
    ],
    mesh=_mesh,
    scratch_types=[
        pltpu.VMEM((CH, K), jnp.int32),
        pltpu.VMEM((K,), jnp.float32),
        pltpu.VMEM_SHARED((NP,), jnp.float32),
        pltpu.SemaphoreType.DMA,
        pltpu.SemaphoreType.DMA,
    ],
)
def _hist_kernel(dst_hbm, ones_hbm, zeros_hbm, h0_hbm, h1_hbm,
                 idx_v, ones_v, hist_sh, hsem0, hsem1):
    c = lax.axis_index("c")
    s = lax.axis_index("s")
    w = c * NS + s
    sl = pl.ds(s * RPT, RPT)
    hsem = (hsem0, hsem1)
    pltpu.sync_copy(zeros_hbm, hist_sh.at[sl])
    pltpu.sync_copy(dst_hbm.at[w], idx_v)
    pltpu.sync_copy(ones_hbm, ones_v)
    plsc.subcore_barrier()

    def body(jj, carry):
        pltpu.async_copy(ones_v, hist_sh.at[idx_v.at[2 * jj]], hsem0,
                         add=True)
        pltpu.async_copy(ones_v, hist_sh.at[idx_v.at[2 * jj + 1]], hsem1,
                         add=True)
        for p in range(2):
            pltpu.make_async_copy(
                ones_v, hist_sh.at[idx_v.at[2 * jj + p]], hsem[p]).wait()
        return carry

    lax.fori_loop(0, CH // 2, body, 0)
    pltpu.sync_copy(ones_v, hist_sh.at[idx_v.at[CH - 1]], add=True)
    plsc.subcore_barrier()

    @pl.when(c == 0)
    def _():
        pltpu.sync_copy(hist_sh.at[sl], h0_hbm.at[sl])

    @pl.when(c == 1)
    def _():
        pltpu.sync_copy(hist_sh.at[sl], h1_hbm.at[sl])


@functools.partial(
    pl.kernel,
    out_type=[
        jax.ShapeDtypeStruct((NP, C), jnp.float32),
        jax.ShapeDtypeStruct((NP, C), jnp.float32),
    ],
    mesh=_mesh,
    scratch_types=[
        pltpu.VMEM((EPW,), jnp.int32),
        pltpu.VMEM((K,), jnp.int32),
        pltpu.VMEM((K,), jnp.int32),
        pltpu.VMEM((K,), jnp.int32),
        pltpu.VMEM((K,), jnp.int32),
        pltpu.VMEM((K,), jnp.int32),
        pltpu.VMEM((K,), jnp.int32),
        pltpu.VMEM((K, C), jnp.float32),
        pltpu.VMEM((K, C), jnp.float32),
        pltpu.VMEM((K, C), jnp.float32),
        pltpu.VMEM_SHARED((NP, C), jnp.float32),
        pltpu.SemaphoreType.DMA,
        pltpu.SemaphoreType.DMA,
        pltpu.SemaphoreType.DMA,
    ],
)
def _edge_kernel(hs_hbm, pidx_hbm, zeros_hbm, acc0_hbm, acc1_hbm,
                 pidx_v, su0, su1, su2, du0, du1, du2,
                 rows0_v, rows1_v, rows2_v, acc_sh, g0sem, g1sem, g2sem):
    c = lax.axis_index("c")
    s = lax.axis_index("s")
    w = c * NS + s
    sl = pl.ds(s * RPT, RPT)
    su = (su0, su1, su2)
    du = (du0, du1, du2)
    rows = (rows0_v, rows1_v, rows2_v)
    gsem = (g0sem, g1sem, g2sem)

    pltpu.sync_copy(zeros_hbm, acc_sh.at[sl])
    pltpu.sync_copy(pidx_hbm.at[w], pidx_v)
    plsc.subcore_barrier()

    def unpack(j, p):
        for t in range(K // 16):
            v = pidx_v[pl.ds(j * K + 16 * t, 16)]
            su[p][pl.ds(16 * t, 16)] = lax.shift_right_logical(v, SHIFT)
            du[p][pl.ds(16 * t, 16)] = lax.bitwise_and(v, MASK)

    unpack(0, 0)
    pltpu.async_copy(hs_hbm.at[su0], rows0_v, g0sem)
    unpack(1, 1)
    pltpu.async_copy(hs_hbm.at[su1], rows1_v, g1sem)

    def step(j, r, prefetch=True):
        pltpu.make_async_copy(hs_hbm.at[su[r]], rows[r], gsem[r]).wait()
        if prefetch:
            q = (r + 2) % 3
            unpack(j + 2, q)
            pltpu.async_copy(hs_hbm.at[su[q]], rows[q], gsem[q])
        pltpu.sync_copy(rows[r], acc_sh.at[du[r]], add=True)

    def body(jj, carry):
        step(3 * jj, 0)
        step(3 * jj + 1, 1)
        step(3 * jj + 2, 2)
        return carry

    lax.fori_loop(0, (CH - 2) // 3, body, 0)
    step(CH - 2, 0, prefetch=False)
    step(CH - 1, 1, prefetch=False)

    plsc.subcore_barrier()

    @pl.when(c == 0)
    def _():
        pltpu.sync_copy(acc_sh.at[sl], acc0_hbm.at[sl])

    @pl.when(c == 1)
    def _():
        pltpu.sync_copy(acc_sh.at[sl], acc1_hbm.at[sl])


def _prep_body(x_ref, w_ref, h0_ref, h1_ref, hs_ref, dis_ref):
    deg = h0_ref[...] + h1_ref[...] + 1.0
    dis = lax.rsqrt(deg)
    h = jnp.dot(x_ref[...], w_ref[...], preferred_element_type=jnp.float32)
    hs_ref[...] = h * dis
    dis_ref[...] = dis


def _final_body(a0_ref, a1_ref, hs_ref, dis_ref, b_ref, out_ref):
    acc = a0_ref[...] + a1_ref[...] + hs_ref[...]
    out_ref[...] = jnp.maximum(acc * dis_ref[...] + b_ref[...], 0.0)


_RB = 1000


def _tc_prep(x, W, h0, h1):
    return pl.pallas_call(
        _prep_body,
        grid=(N // _RB,),
        in_specs=[
            pl.BlockSpec((_RB, C), lambda i: (i, 0)),
            pl.BlockSpec((C, C), lambda i: (0, 0)),
            pl.BlockSpec((_RB, 1), lambda i: (i, 0)),
            pl.BlockSpec((_RB, 1), lambda i: (i, 0)),
        ],
        out_specs=[
            pl.BlockSpec((_RB, C), lambda i: (i, 0)),
            pl.BlockSpec((_RB, 1), lambda i: (i, 0)),
        ],
        out_shape=[
            jax.ShapeDtypeStruct((N, C), jnp.float32),
            jax.ShapeDtypeStruct((N, 1), jnp.float32),
        ],
    )(x, W, h0, h1)


def _tc_final(a0, a1, hs, dis, b):
    return pl.pallas_call(
        _final_body,
        grid=(N // _RB,),
        in_specs=[
            pl.BlockSpec((_RB, C), lambda i: (i, 0)),
            pl.BlockSpec((_RB, C), lambda i: (i, 0)),
            pl.BlockSpec((_RB, C), lambda i: (i, 0)),
            pl.BlockSpec((_RB, 1), lambda i: (i, 0)),
            pl.BlockSpec((1, C), lambda i: (0, 0)),
        ],
        out_specs=pl.BlockSpec((_RB, C), lambda i: (i, 0)),
        out_shape=jax.ShapeDtypeStruct((N, C), jnp.float32),
    )(a0, a1, hs, dis, b)


def kernel(x, edge_index, W, b):
    src = edge_index[0]
    dst = edge_index[1]
    dst3d = dst.reshape(NW, CH, K)
    pidx2d = (lax.shift_left(src, SHIFT) | dst).reshape(NW, EPW)
    ones_k = jnp.ones((K,), jnp.float32)
    zeros1 = jnp.zeros((RPT,), jnp.float32)
    zeros128 = jnp.zeros((RPT, C), jnp.float32)

    h0, h1 = _hist_kernel(dst3d, ones_k, zeros1)
    hs, dis = _tc_prep(x, W, h0.reshape(NP, 1), h1.reshape(NP, 1))
    acc0, acc1 = _edge_kernel(hs, pidx2d, zeros128)
    return _tc_final(acc0, acc1, hs, dis, b.reshape(1, C))


# speedup vs baseline: 2.6519x; 1.0089x over previous
import functools

import jax
import jax.numpy as jnp
from jax import lax
from jax.experimental import pallas as pl
from jax.experimental.pallas import tpu as pltpu
from jax.experimental.pallas import tpu_sc as plsc

N = 10000
E = 320000
C = 128

NC = 2
NS = 16
NW = NC * NS
EPW = E // NW
K = 80
CH = EPW // K
NP = 10240
RPT = NP // NS
SHIFT = 14
MASK = (1 << SHIFT) - 1

_mesh = plsc.VectorSubcoreMesh(core_axis_name="c", subcore_axis_name="s")


@functools.partial(
    pl.kernel,
    out_type=[
        jax.ShapeDtypeStruct((NP,), jnp.float32),
        jax.ShapeDtypeStruct((NP,), jnp.float32),
    ],
    mesh=_mesh,
    scratch_types=[
        pltpu.VMEM((CH, K), jnp.int32),
        pltpu.VMEM((K,), jnp.float32),
        pltpu.VMEM_SHARED((NP,), jnp.float32),
        pltpu.SemaphoreType.DMA,
        pltpu.SemaphoreType.DMA,
    ],
)
def _hist_kernel(dst_hbm, ones_hbm, zeros_hbm, h0_hbm, h1_hbm,
                 idx_v, ones_v, hist_sh, hsem0, hsem1):
    c = lax.axis_index("c")
    s = lax.axis_index("s")
    w = c * NS + s
    sl = pl.ds(s * RPT, RPT)
    hsem = (hsem0, hsem1)
    pltpu.sync_copy(zeros_hbm, hist_sh.at[sl])
    pltpu.sync_copy(dst_hbm.at[w], idx_v)
    pltpu.sync_copy(ones_hbm, ones_v)
    plsc.subcore_barrier()

    def body(jj, carry):
        pltpu.async_copy(ones_v, hist_sh.at[idx_v.at[2 * jj]], hsem0,
                         add=True)
        pltpu.async_copy(ones_v, hist_sh.at[idx_v.at[2 * jj + 1]], hsem1,
                         add=True)
        for p in range(2):
            pltpu.make_async_copy(
                ones_v, hist_sh.at[idx_v.at[2 * jj + p]], hsem[p]).wait()
        return carry

    lax.fori_loop(0, CH // 2, body, 0)
    pltpu.sync_copy(ones_v, hist_sh.at[idx_v.at[CH - 1]], add=True)
    plsc.subcore_barrier()

    @pl.when(c == 0)
    def _():
        pltpu.sync_copy(hist_sh.at[sl], h0_hbm.at[sl])

    @pl.when(c == 1)
    def _():
        pltpu.sync_copy(hist_sh.at[sl], h1_hbm.at[sl])


@functools.partial(
    pl.kernel,
    out_type=[
        jax.ShapeDtypeStruct((NP, C), jnp.float32),
        jax.ShapeDtypeStruct((NP, C), jnp.float32),
    ],
    mesh=_mesh,
    scratch_types=[
        pltpu.VMEM((EPW,), jnp.int32),
        pltpu.VMEM((K,), jnp.int32),
        pltpu.VMEM((K,), jnp.int32),
        pltpu.VMEM((K,), jnp.int32),
        pltpu.VMEM((K,), jnp.int32),
        pltpu.VMEM((K,), jnp.int32),
        pltpu.VMEM((K,), jnp.int32),
        pltpu.VMEM((K, C), jnp.float32),
        pltpu.VMEM((K, C), jnp.float32),
        pltpu.VMEM((K, C), jnp.float32),
        pltpu.VMEM_SHARED((NP, C), jnp.float32),
        pltpu.SemaphoreType.DMA,
        pltpu.SemaphoreType.DMA,
        pltpu.SemaphoreType.DMA,
    ],
)
def _edge_kernel(hs_hbm, pidx_hbm, zeros_hbm, acc0_hbm, acc1_hbm,
                 pidx_v, su0, su1, su2, du0, du1, du2,
                 rows0_v, rows1_v, rows2_v, acc_sh, g0sem, g1sem, g2sem):
    c = lax.axis_index("c")
    s = lax.axis_index("s")
    w = c * NS + s
    sl = pl.ds(s * RPT, RPT)
    su = (su0, su1, su2)
    du = (du0, du1, du2)
    rows = (rows0_v, rows1_v, rows2_v)
    gsem = (g0sem, g1sem, g2sem)

    @pl.when(c == 0)
    def _():
        pltpu.sync_copy(hs_hbm.at[sl], acc_sh.at[sl])

    @pl.when(c == 1)
    def _():
        pltpu.sync_copy(zeros_hbm, acc_sh.at[sl])

    pltpu.sync_copy(pidx_hbm.at[w], pidx_v)
    plsc.subcore_barrier()

    def unpack(j, p):
        for t in range(K // 16):
            v = pidx_v[pl.ds(j * K + 16 * t, 16)]
            su[p][pl.ds(16 * t, 16)] = lax.shift_right_logical(v, SHIFT)
            du[p][pl.ds(16 * t, 16)] = lax.bitwise_and(v, MASK)

    unpack(0, 0)
    pltpu.async_copy(hs_hbm.at[su0], rows0_v, g0sem)
    unpack(1, 1)
    pltpu.async_copy(hs_hbm.at[su1], rows1_v, g1sem)

    def step(j, r, prefetch=True):
        pltpu.make_async_copy(hs_hbm.at[su[r]], rows[r], gsem[r]).wait()
        if prefetch:
            q = (r + 2) % 3
            unpack(j + 2, q)
            pltpu.async_copy(hs_hbm.at[su[q]], rows[q], gsem[q])
        pltpu.sync_copy(rows[r], acc_sh.at[du[r]], add=True)

    def body(jj, carry):
        step(3 * jj, 0)
        step(3 * jj + 1, 1)
        step(3 * jj + 2, 2)
        return carry

    lax.fori_loop(0, (CH - 2) // 3, body, 0)
    step(CH - 2, 0, prefetch=False)
    step(CH - 1, 1, prefetch=False)

    plsc.subcore_barrier()

    @pl.when(c == 0)
    def _():
        pltpu.sync_copy(acc_sh.at[sl], acc0_hbm.at[sl])

    @pl.when(c == 1)
    def _():
        pltpu.sync_copy(acc_sh.at[sl], acc1_hbm.at[sl])


def _prep_body(x_ref, w_ref, h0_ref, h1_ref, hs_ref, dis_ref):
    deg = h0_ref[...] + h1_ref[...] + 1.0
    dis = lax.rsqrt(deg)
    h = jnp.dot(x_ref[...], w_ref[...], preferred_element_type=jnp.float32)
    hs_ref[...] = h * dis
    dis_ref[...] = dis


def _final_body(a0_ref, a1_ref, dis_ref, b_ref, out_ref):
    acc = a0_ref[...] + a1_ref[...]
    out_ref[...] = jnp.maximum(acc * dis_ref[...] + b_ref[...], 0.0)


_RB = 1000
_RBP = 1024


def _tc_prep(x, W, h0, h1):
    return pl.pallas_call(
        _prep_body,
        grid=(NP // _RBP,),
        in_specs=[
            pl.BlockSpec((_RBP, C), lambda i: (i, 0)),
            pl.BlockSpec((C, C), lambda i: (0, 0)),
            pl.BlockSpec((_RBP, 1), lambda i: (i, 0)),
            pl.BlockSpec((_RBP, 1), lambda i: (i, 0)),
        ],
        out_specs=[
            pl.BlockSpec((_RBP, C), lambda i: (i, 0)),
            pl.BlockSpec((_RBP, 1), lambda i: (i, 0)),
        ],
        out_shape=[
            jax.ShapeDtypeStruct((NP, C), jnp.float32),
            jax.ShapeDtypeStruct((NP, 1), jnp.float32),
        ],
    )(x, W, h0, h1)


def _tc_final(a0, a1, dis, b):
    return pl.pallas_call(
        _final_body,
        grid=(N // _RB,),
        in_specs=[
            pl.BlockSpec((_RB, C), lambda i: (i, 0)),
            pl.BlockSpec((_RB, C), lambda i: (i, 0)),
            pl.BlockSpec((_RB, 1), lambda i: (i, 0)),
            pl.BlockSpec((1, C), lambda i: (0, 0)),
        ],
        out_specs=pl.BlockSpec((_RB, C), lambda i: (i, 0)),
        out_shape=jax.ShapeDtypeStruct((N, C), jnp.float32),
    )(a0, a1, dis, b)


def kernel(x, edge_index, W, b):
    src = edge_index[0]
    dst = edge_index[1]
    dst3d = dst.reshape(NW, CH, K)
    pidx2d = (lax.shift_left(src, SHIFT) | dst).reshape(NW, EPW)
    ones_k = jnp.ones((K,), jnp.float32)
    zeros1 = jnp.zeros((RPT,), jnp.float32)
    zeros128 = jnp.zeros((RPT, C), jnp.float32)

    h0, h1 = _hist_kernel(dst3d, ones_k, zeros1)
    hs, dis = _tc_prep(x, W, h0.reshape(NP, 1), h1.reshape(NP, 1))
    acc0, acc1 = _edge_kernel(hs, pidx2d, zeros128)
    return _tc_final(acc0, acc1, dis, b.reshape(1, C))
